# baseline (speedup 1.0000x reference)
import functools

import jax
import jax.numpy as jnp
from jax import lax
from jax.experimental import pallas as pl
from jax.experimental.pallas import tpu as pltpu

_D = 16
_EB = 8000
_NB = 2000
_INTERPRET = False


def _lrelu(x, s):
    return jnp.where(x >= 0, x, s * x)


def _dot(a, b):
    return jnp.dot(a, b, preferred_element_type=jnp.float32)


def _edge_body(u, W1, b1, W2, b2, hs, ht, xe, out, sums):
    i = pl.program_id(0)
    n = hs.shape[0]
    h = jnp.concatenate(
        [hs[...], ht[...], xe[...], jnp.broadcast_to(u[...], (n, _D))], axis=1)
    y = _lrelu(_dot(h, W1[...]) + b1[...], 0.1)
    z = _dot(y, W2[...]) + b2[...]
    out[...] = z
    ps = jnp.concatenate(
        [jnp.sum(z, 0, keepdims=True), jnp.sum(z * z, 0, keepdims=True)], axis=0)

    @pl.when(i == 0)
    def _():
        sums[...] = ps

    @pl.when(i != 0)
    def _():
        sums[...] = sums[...] + ps


def _msg_s_body(scale, shift, W1, b1, W2, b2, elin, ht, xen_out, msg_out):
    xen = elin[...] * scale[...] + shift[...]
    h = jnp.concatenate([ht[...], xen], axis=1)
    m = _lrelu(_dot(h, W1[...]) + b1[...], 0.1)
    msg_out[...] = _dot(m, W2[...]) + b2[...]
    xen_out[...] = xen


def _stats_body(cnt, S1, S2, mean_out, std_out):
    c = cnt[...]
    denom = jnp.maximum(c[0, :, :1] + c[1, :, :1], 1.0)
    s1 = S1[...]
    s2 = S2[...]
    mu = (s1[0] + s1[1]) / denom
    m2 = (s2[0] + s2[1]) / denom
    var = _lrelu(m2 - mu * mu, 0.01)
    mean_out[...] = mu
    std_out[...] = jnp.sqrt(var + 1e-6)


def _src_node_body(u, W1, b1, W2, b2, xs, cnt, mean, std, C3, C4, out, sums):
    i = pl.program_id(0)
    n = xs.shape[0]
    c = cnt[...]
    denom = jnp.maximum(c[0, :, :1] + c[1, :, :1], 1.0)
    mu = mean[...]
    sd = std[...]
    c3 = C3[...]
    c4 = C4[...]
    std3 = sd * sd * sd
    skew = (c3[0] + c3[1]) / denom / std3
    kurt = (c4[0] + c4[1]) / denom / (std3 * sd)
    h = jnp.concatenate(
        [xs[...], mu, sd, skew, kurt, jnp.broadcast_to(u[...], (n, _D))], axis=1)
    y = _lrelu(_dot(h, W1[...]) + b1[...], 0.1)
    z = _dot(y, W2[...]) + b2[...]
    out[...] = z
    ps = jnp.concatenate(
        [jnp.sum(z, 0, keepdims=True), jnp.sum(z * z, 0, keepdims=True)], axis=0)

    @pl.when(i == 0)
    def _():
        sums[...] = ps

    @pl.when(i != 0)
    def _():
        sums[...] = sums[...] + ps


def _msg_t_body(W1, b1, W2, b2, hss, xen, msg_out):
    h = jnp.concatenate([hss[...], xen[...]], axis=1)
    m = _lrelu(_dot(h, W1[...]) + b1[...], 0.1)
    msg_out[...] = _dot(m, W2[...]) + b2[...]


def _tgt_node_body(u, W1, b1, W2, b2, xt, agg, out, sums):
    i = pl.program_id(0)
    n = xt.shape[0]
    a = agg[...]
    h = jnp.concatenate(
        [xt[...], a[0] + a[1], jnp.broadcast_to(u[...], (n, _D))], axis=1)
    y = _lrelu(_dot(h, W1[...]) + b1[...], 0.1)
    z = _dot(y, W2[...]) + b2[...]
    out[...] = z
    ps = jnp.concatenate(
        [jnp.sum(z, 0, keepdims=True), jnp.sum(z * z, 0, keepdims=True)], axis=0)

    @pl.when(i == 0)
    def _():
        sums[...] = ps

    @pl.when(i != 0)
    def _():
        sums[...] = sums[...] + ps


def _fin_body(scale, shift, lin, xn_out, colsum_out):
    i = pl.program_id(0)
    xn = lin[...] * scale[...] + shift[...]
    xn_out[...] = xn
    cs = jnp.sum(xn, 0, keepdims=True)

    @pl.when(i == 0)
    def _():
        colsum_out[...] = cs

    @pl.when(i != 0)
    def _():
        colsum_out[...] = colsum_out[...] + cs


def _global_body(ns, nt, s_sum, t_sum, x_u, Wg1, bg1, Wg2, bg2, g_u, xun_out):
    h_u = jnp.concatenate(
        [x_u[...], s_sum[...] / float(ns), t_sum[...] / float(nt)], axis=1)
    y = _lrelu(_dot(h_u, Wg1[...]) + bg1[...], 0.1)
    g = _dot(y, Wg2[...]) + bg2[...]
    xun_out[...] = g / jnp.sqrt(
        jnp.mean(g * g, axis=-1, keepdims=True) + 1.1920929e-07) * g_u[...]


def _full(shape):
    return pl.BlockSpec(shape, lambda i: tuple(0 for _ in shape))


def _rows(bs, w):
    return pl.BlockSpec((bs, w), lambda i: (i, 0))


def _rows3(bs, w):
    return pl.BlockSpec((2, bs, w), lambda i: (0, i, 0))


def _sum_spec():
    return pl.BlockSpec((2, _D), lambda i: (0, 0))


def _tc(body, grid, in_specs, out_specs, out_shape, args):
    return pl.pallas_call(
        body, grid=grid, in_specs=in_specs, out_specs=out_specs,
        out_shape=out_shape, interpret=_INTERPRET)(*args)


def kernel(x_s, x_t, x_e, x_u, edge_index, params):
    p = params
    E = x_e.shape[0]
    NS = x_s.shape[0]
    NT = x_t.shape[0]
    src = edge_index[0]
    tgt = edge_index[1]

    hs = jnp.take(x_s, src, axis=0)
    ht = jnp.take(x_t, tgt, axis=0)

    r2 = lambda a: a.reshape(1, -1)
    ge = E // _EB
    elin, esums = _tc(
        _edge_body, (ge,),
        [_full((1, _D)), _full((64, 64)), _full((1, 64)), _full((64, _D)),
         _full((1, _D)), _rows(_EB, _D), _rows(_EB, _D), _rows(_EB, _D)],
        [_rows(_EB, _D), _sum_spec()],
        [jax.ShapeDtypeStruct((E, _D), jnp.float32),
         jax.ShapeDtypeStruct((2, _D), jnp.float32)],
        (x_u, p['We1'], r2(p['be1']), p['We2'], r2(p['be2']), hs, ht, x_e))
    m_e = esums[0] / E
    v_e = esums[1] / E - m_e * m_e
    sc_e = p['g_e'] / jnp.sqrt(v_e + 1e-5)
    sh_e = p['b_e'] - m_e * sc_e

    xen, msg = _tc(
        _msg_s_body, (ge,),
        [_full((1, _D)), _full((1, _D)), _full((32, 32)), _full((1, 32)),
         _full((32, 32)), _full((1, 32)), _rows(_EB, _D), _rows(_EB, _D)],
        [_rows(_EB, _D), _rows(_EB, 32)],
        [jax.ShapeDtypeStruct((E, _D), jnp.float32),
         jax.ShapeDtypeStruct((E, 32), jnp.float32)],
        (r2(sc_e), r2(sh_e), p['Ws11'], r2(p['bs11']), p['Ws12'], r2(p['bs12']),
         elin, ht))

    ones = jnp.ones((E,), jnp.float32)
    cnt = jax.ops.segment_sum(ones, src, num_segments=NS)
    cnt2 = jnp.broadcast_to(cnt[None, :, None], (1, NS, _D))
    cnt2 = jnp.concatenate([cnt2, jnp.zeros((1, NS, _D), jnp.float32)], 0)
    S1 = jax.ops.segment_sum(msg, src, num_segments=NS)
    S2 = jax.ops.segment_sum(msg * msg, src, num_segments=NS)
    pad0 = lambda a: jnp.stack([a, jnp.zeros_like(a)], 0)
    S1, S2 = pad0(S1), pad0(S2)

    gn = NS // _NB
    mean, std = _tc(
        _stats_body, (gn,),
        [_rows3(_NB, _D), _rows3(_NB, 32), _rows3(_NB, 32)],
        [_rows(_NB, 32), _rows(_NB, 32)],
        [jax.ShapeDtypeStruct((NS, 32), jnp.float32),
         jax.ShapeDtypeStruct((NS, 32), jnp.float32)],
        (cnt2, S1, S2))

    cmsg = msg - jnp.take(mean, src, axis=0)
    C3 = pad0(jax.ops.segment_sum(cmsg ** 3, src, num_segments=NS))
    C4 = pad0(jax.ops.segment_sum(cmsg ** 4, src, num_segments=NS))

    slin, ssums = _tc(
        _src_node_body, (gn,),
        [_full((1, _D)), _full((160, 160)), _full((1, 160)), _full((160, _D)),
         _full((1, _D)), _rows(_NB, _D), _rows3(_NB, _D),
         _rows(_NB, 32), _rows(_NB, 32), _rows3(_NB, 32), _rows3(_NB, 32)],
        [_rows(_NB, _D), _sum_spec()],
        [jax.ShapeDtypeStruct((NS, _D), jnp.float32),
         jax.ShapeDtypeStruct((2, _D), jnp.float32)],
        (x_u, p['Ws21'], r2(p['bs21']), p['Ws22'], r2(p['bs22']),
         x_s, cnt2, mean, std, C3, C4))
    m_s = ssums[0] / NS
    v_s = ssums[1] / NS - m_s * m_s
    sc_s = p['g_s'] / jnp.sqrt(v_s + 1e-5)
    sh_s = p['b_s'] - m_s * sc_s

    xsn, s_sum = _tc(
        _fin_body, (gn,),
        [_full((1, _D)), _full((1, _D)), _rows(_NB, _D)],
        [_rows(_NB, _D), pl.BlockSpec((1, _D), lambda i: (0, 0))],
        [jax.ShapeDtypeStruct((NS, _D), jnp.float32),
         jax.ShapeDtypeStruct((1, _D), jnp.float32)],
        (r2(sc_s), r2(sh_s), slin))

    hss = jnp.take(xsn, src, axis=0)

    (msgt,) = _tc(
        _msg_t_body, (ge,),
        [_full((32, 32)), _full((1, 32)), _full((32, 32)), _full((1, 32)),
         _rows(_EB, _D), _rows(_EB, _D)],
        [_rows(_EB, 32)],
        [jax.ShapeDtypeStruct((E, 32), jnp.float32)],
        (p['Wt11'], r2(p['bt11']), p['Wt12'], r2(p['bt12']), hss, xen))

    agg = pad0(jax.ops.segment_sum(msgt, tgt, num_segments=NT))

    gt = NT // _NB
    tlin, tsums = _tc(
        _tgt_node_body, (gt,),
        [_full((1, _D)), _full((64, 64)), _full((1, 64)), _full((64, _D)),
         _full((1, _D)), _rows(_NB, _D), _rows3(_NB, 32)],
        [_rows(_NB, _D), _sum_spec()],
        [jax.ShapeDtypeStruct((NT, _D), jnp.float32),
         jax.ShapeDtypeStruct((2, _D), jnp.float32)],
        (x_u, p['Wt21'], r2(p['bt21']), p['Wt22'], r2(p['bt22']), x_t, agg))
    m_t = tsums[0] / NT
    v_t = tsums[1] / NT - m_t * m_t
    sc_t = p['g_t'] / jnp.sqrt(v_t + 1e-5)
    sh_t = p['b_t'] - m_t * sc_t

    xtn, t_sum = _tc(
        _fin_body, (gt,),
        [_full((1, _D)), _full((1, _D)), _rows(_NB, _D)],
        [_rows(_NB, _D), pl.BlockSpec((1, _D), lambda i: (0, 0))],
        [jax.ShapeDtypeStruct((NT, _D), jnp.float32),
         jax.ShapeDtypeStruct((1, _D), jnp.float32)],
        (r2(sc_t), r2(sh_t), tlin))

    (xun,) = _tc(
        functools.partial(_global_body, NS, NT), (1,),
        [_full((1, _D)), _full((1, _D)), _full((1, _D)), _full((48, 48)),
         _full((1, 48)), _full((48, _D)), _full((1, _D)), _full((1, _D))],
        [_full((1, _D))],
        [jax.ShapeDtypeStruct((1, _D), jnp.float32)],
        (s_sum, t_sum, x_u, p['Wg1'], r2(p['bg1']), p['Wg2'], r2(p['bg2']),
         r2(p['g_u'])))

    return (edge_index, xsn, xtn, xen, xun)


# speedup vs baseline: 3.0467x; 3.0467x over previous
import functools

import jax
import jax.numpy as jnp
from jax import lax
from jax.experimental import pallas as pl
from jax.experimental.pallas import tpu as pltpu
from jax.experimental.pallas import tpu_sc as plsc

_D = 16
_NC = 2
_NSUB = 16
_NW = _NC * _NSUB
_EB = 8000
_NB = 2000
_INTERPRET = False


def _lrelu(x, s):
    return jnp.where(x >= 0, x, s * x)


def _dot(a, b):
    return jnp.dot(a, b, preferred_element_type=jnp.float32)


def _edge_body(u, W1, b1, W2, b2, hs, ht, xe, out, sums):
    i = pl.program_id(0)
    n = hs.shape[0]
    h = jnp.concatenate(
        [hs[...], ht[...], xe[...], jnp.broadcast_to(u[...], (n, _D))], axis=1)
    y = _lrelu(_dot(h, W1[...]) + b1[...], 0.1)
    z = _dot(y, W2[...]) + b2[...]
    out[...] = z
    ps = jnp.concatenate(
        [jnp.sum(z, 0, keepdims=True), jnp.sum(z * z, 0, keepdims=True)], axis=0)

    @pl.when(i == 0)
    def _():
        sums[...] = ps

    @pl.when(i != 0)
    def _():
        sums[...] = sums[...] + ps


def _msg_s_body(scale, shift, W1, b1, W2, b2, elin, ht, xen_out, msg_out):
    xen = elin[...] * scale[...] + shift[...]
    h = jnp.concatenate([ht[...], xen], axis=1)
    m = _lrelu(_dot(h, W1[...]) + b1[...], 0.1)
    msg_out[...] = _dot(m, W2[...]) + b2[...]
    xen_out[...] = xen


def _stats_body(cnt, S1, S2, mean_out, std_out):
    c = cnt[...]
    denom = jnp.maximum(c[0, :, :1] + c[1, :, :1], 1.0)
    s1 = S1[...]
    s2 = S2[...]
    mu = (s1[0] + s1[1]) / denom
    m2 = (s2[0] + s2[1]) / denom
    var = _lrelu(m2 - mu * mu, 0.01)
    mean_out[...] = mu
    std_out[...] = jnp.sqrt(var + 1e-6)


def _src_node_body(u, W1, b1, W2, b2, xs, cnt, mean, std, C3, C4, out, sums):
    i = pl.program_id(0)
    n = xs.shape[0]
    c = cnt[...]
    denom = jnp.maximum(c[0, :, :1] + c[1, :, :1], 1.0)
    mu = mean[...]
    sd = std[...]
    c3 = C3[...]
    c4 = C4[...]
    std3 = sd * sd * sd
    skew = (c3[0] + c3[1]) / denom / std3
    kurt = (c4[0] + c4[1]) / denom / (std3 * sd)
    h = jnp.concatenate(
        [xs[...], mu, sd, skew, kurt, jnp.broadcast_to(u[...], (n, _D))], axis=1)
    y = _lrelu(_dot(h, W1[...]) + b1[...], 0.1)
    z = _dot(y, W2[...]) + b2[...]
    out[...] = z
    ps = jnp.concatenate(
        [jnp.sum(z, 0, keepdims=True), jnp.sum(z * z, 0, keepdims=True)], axis=0)

    @pl.when(i == 0)
    def _():
        sums[...] = ps

    @pl.when(i != 0)
    def _():
        sums[...] = sums[...] + ps


def _msg_t_body(W1, b1, W2, b2, hss, xen, msg_out):
    h = jnp.concatenate([hss[...], xen[...]], axis=1)
    m = _lrelu(_dot(h, W1[...]) + b1[...], 0.1)
    msg_out[...] = _dot(m, W2[...]) + b2[...]


def _tgt_node_body(u, W1, b1, W2, b2, xt, agg, out, sums):
    i = pl.program_id(0)
    n = xt.shape[0]
    a = agg[...]
    h = jnp.concatenate(
        [xt[...], a[0] + a[1], jnp.broadcast_to(u[...], (n, _D))], axis=1)
    y = _lrelu(_dot(h, W1[...]) + b1[...], 0.1)
    z = _dot(y, W2[...]) + b2[...]
    out[...] = z
    ps = jnp.concatenate(
        [jnp.sum(z, 0, keepdims=True), jnp.sum(z * z, 0, keepdims=True)], axis=0)

    @pl.when(i == 0)
    def _():
        sums[...] = ps

    @pl.when(i != 0)
    def _():
        sums[...] = sums[...] + ps


def _fin_body(scale, shift, lin, xn_out, colsum_out):
    i = pl.program_id(0)
    xn = lin[...] * scale[...] + shift[...]
    xn_out[...] = xn
    cs = jnp.sum(xn, 0, keepdims=True)

    @pl.when(i == 0)
    def _():
        colsum_out[...] = cs

    @pl.when(i != 0)
    def _():
        colsum_out[...] = colsum_out[...] + cs


def _global_body(ns, nt, s_sum, t_sum, x_u, Wg1, bg1, Wg2, bg2, g_u, xun_out):
    h_u = jnp.concatenate(
        [x_u[...], s_sum[...] / float(ns), t_sum[...] / float(nt)], axis=1)
    y = _lrelu(_dot(h_u, Wg1[...]) + bg1[...], 0.1)
    g = _dot(y, Wg2[...]) + bg2[...]
    xun_out[...] = g / jnp.sqrt(
        jnp.mean(g * g, axis=-1, keepdims=True) + 1.1920929e-07) * g_u[...]


def _sc_mesh():
    return plsc.VectorSubcoreMesh(core_axis_name="c", subcore_axis_name="s")


def _sc_gather(tables_and_idx):
    n = len(tables_and_idx)
    E = tables_and_idx[0][1].shape[0]
    D = tables_and_idx[0][0].shape[1]
    G = E // 128
    gbase, grem = G // _NW, G % _NW

    out_type = [jax.ShapeDtypeStruct((E, D), jnp.float32) for _ in range(n)]
    scratch = ([pltpu.VMEM((1, 128), jnp.int32) for _ in range(n)]
               + [pltpu.VMEM((128, D), jnp.float32) for _ in range(n)]
               + [pltpu.SemaphoreType.DMA for _ in range(n)])

    @functools.partial(
        pl.kernel, mesh=_sc_mesh(), out_type=out_type, scratch_types=scratch,
        compiler_params=pltpu.CompilerParams(use_tc_tiling_on_sc=False))
    def k(*refs):
        tabs = refs[:n]
        idxs = refs[n:2 * n]
        outs = refs[2 * n:3 * n]
        ivs = refs[3 * n:4 * n]
        rvs = refs[4 * n:5 * n]
        sems = refs[5 * n:6 * n]
        w = lax.axis_index("s") * _NC + lax.axis_index("c")
        g0 = w * gbase + jnp.minimum(w, grem)
        g1 = g0 + gbase + jnp.where(w < grem, 1, 0)

        def body(g, carry):
            off = g * 128
            for j in range(n):
                pltpu.sync_copy(idxs[j].at[pl.ds(off, 128)], ivs[j].at[0])
            cps = [pltpu.async_copy(tabs[j].at[ivs[j].at[0]], rvs[j], sems[j])
                   for j in range(n)]
            for cp in cps:
                cp.wait()
            for j in range(n):
                pltpu.sync_copy(rvs[j], outs[j].at[pl.ds(off, 128)])
            return carry

        lax.fori_loop(g0, g1, body, 0)

    flat = [t for t, _ in tables_and_idx] + [i for _, i in tables_and_idx]
    res = k(*flat)
    return list(res) if isinstance(res, (tuple, list)) else [res]


def _sc_segsum(idx, ns, width, power, data=None, mean=None):
    E = idx.shape[0]
    G = E // 128
    gbase, grem = G // _NW, G % _NW
    rps = ns // _NSUB
    zr = 125
    assert rps % zr == 0
    nvec = 128 * width // 16
    cpr = width // 16

    inputs = [idx] + ([data] if data is not None else []) \
                   + ([mean] if mean is not None else [])
    scratch = [pltpu.VMEM((1, 128), jnp.int32),
               pltpu.VMEM((128, width), jnp.float32),
               pltpu.VMEM((128, width), jnp.float32),
               pltpu.VMEM((128, width), jnp.float32),
               pltpu.VMEM((zr, width), jnp.float32),
               pltpu.VMEM_SHARED((ns, width), jnp.float32),
               pltpu.SemaphoreType.DMA]

    @functools.partial(
        pl.kernel, mesh=_sc_mesh(),
        out_type=jax.ShapeDtypeStruct((2, ns, width), jnp.float32),
        scratch_types=scratch,
        compiler_params=pltpu.CompilerParams(use_tc_tiling_on_sc=False))
    def k(*refs):
        pos = len(inputs)
        idx_h = refs[0]
        data_h = refs[1] if data is not None else None
        mean_h = refs[2] if mean is not None else None
        out_h = refs[pos]
        iv, dv, pv, mv, zb, acc, sem = refs[pos + 1:]
        c = lax.axis_index("c")
        s = lax.axis_index("s")
        w = s * _NC + c
        g0 = w * gbase + jnp.minimum(w, grem)
        g1 = g0 + gbase + jnp.where(w < grem, 1, 0)

        zeros16 = jnp.zeros((16,), jnp.float32)
        ones16 = jnp.ones((16,), jnp.float32)

        def zfill(i, carry):
            r = i // cpr
            col = (i % cpr) * 16
            zb[r, pl.ds(col, 16)] = zeros16
            return carry

        lax.fori_loop(0, zr * cpr, zfill, 0)

        if power == 0:
            def ofill(i, carry):
                r = i // cpr
                col = (i % cpr) * 16
                pv[r, pl.ds(col, 16)] = ones16
                return carry

            lax.fori_loop(0, nvec, ofill, 0)

        def zacc(i, carry):
            pltpu.sync_copy(zb, acc.at[pl.ds(s * rps + i * zr, zr)])
            return carry

        lax.fori_loop(0, rps // zr, zacc, 0)
        plsc.subcore_barrier()

        def body(g, carry):
            off = g * 128
            pltpu.sync_copy(idx_h.at[pl.ds(off, 128)], iv.at[0])
            if data is not None:
                pltpu.sync_copy(data_h.at[pl.ds(off, 128)], dv)
            if mean is not None:
                pltpu.async_copy(mean_h.at[iv.at[0]], mv, sem).wait()
            if power >= 1 and not (power == 1 and mean is None):
                def pw(i, cc):
                    r = i // cpr
                    col = (i % cpr) * 16
                    x = dv[r, pl.ds(col, 16)]
                    if mean is not None:
                        x = x - mv[r, pl.ds(col, 16)]
                    if power == 1:
                        y = x
                    elif power == 2:
                        y = x * x
                    elif power == 3:
                        y = x * x * x
                    else:
                        x2 = x * x
                        y = x2 * x2
                    pv[r, pl.ds(col, 16)] = y
                    return cc

                lax.fori_loop(0, nvec, pw, 0)
            sbuf = dv if (power == 1 and mean is None) else pv
            pltpu.sync_copy(sbuf, acc.at[iv.at[0]], add=True)
            return carry

        lax.fori_loop(g0, g1, body, 0)
        plsc.subcore_barrier()
        pltpu.sync_copy(acc.at[pl.ds(s * rps, rps)],
                        out_h.at[c, pl.ds(s * rps, rps)])

    return k(*inputs)


def _full(shape):
    return pl.BlockSpec(shape, lambda i: tuple(0 for _ in shape))


def _rows(bs, w):
    return pl.BlockSpec((bs, w), lambda i: (i, 0))


def _rows3(bs, w):
    return pl.BlockSpec((2, bs, w), lambda i: (0, i, 0))


def _sum_spec():
    return pl.BlockSpec((2, _D), lambda i: (0, 0))


def _tc(body, grid, in_specs, out_specs, out_shape, args):
    return pl.pallas_call(
        body, grid=grid, in_specs=in_specs, out_specs=out_specs,
        out_shape=out_shape, interpret=_INTERPRET)(*args)


def kernel(x_s, x_t, x_e, x_u, edge_index, params):
    p = params
    E = x_e.shape[0]
    NS = x_s.shape[0]
    NT = x_t.shape[0]
    src = edge_index[0]
    tgt = edge_index[1]

    hs, ht = _sc_gather([(x_s, src), (x_t, tgt)])

    r2 = lambda a: a.reshape(1, -1)
    ge = E // _EB
    elin, esums = _tc(
        _edge_body, (ge,),
        [_full((1, _D)), _full((64, 64)), _full((1, 64)), _full((64, _D)),
         _full((1, _D)), _rows(_EB, _D), _rows(_EB, _D), _rows(_EB, _D)],
        [_rows(_EB, _D), _sum_spec()],
        [jax.ShapeDtypeStruct((E, _D), jnp.float32),
         jax.ShapeDtypeStruct((2, _D), jnp.float32)],
        (x_u, p['We1'], r2(p['be1']), p['We2'], r2(p['be2']), hs, ht, x_e))
    m_e = esums[0] / E
    v_e = esums[1] / E - m_e * m_e
    sc_e = p['g_e'] / jnp.sqrt(v_e + 1e-5)
    sh_e = p['b_e'] - m_e * sc_e

    xen, msg = _tc(
        _msg_s_body, (ge,),
        [_full((1, _D)), _full((1, _D)), _full((32, 32)), _full((1, 32)),
         _full((32, 32)), _full((1, 32)), _rows(_EB, _D), _rows(_EB, _D)],
        [_rows(_EB, _D), _rows(_EB, 32)],
        [jax.ShapeDtypeStruct((E, _D), jnp.float32),
         jax.ShapeDtypeStruct((E, 32), jnp.float32)],
        (r2(sc_e), r2(sh_e), p['Ws11'], r2(p['bs11']), p['Ws12'], r2(p['bs12']),
         elin, ht))

    cnt2 = _sc_segsum(src, NS, _D, 0)
    S1 = _sc_segsum(src, NS, 32, 1, data=msg)
    S2 = _sc_segsum(src, NS, 32, 2, data=msg)

    gn = NS // _NB
    mean, std = _tc(
        _stats_body, (gn,),
        [_rows3(_NB, _D), _rows3(_NB, 32), _rows3(_NB, 32)],
        [_rows(_NB, 32), _rows(_NB, 32)],
        [jax.ShapeDtypeStruct((NS, 32), jnp.float32),
         jax.ShapeDtypeStruct((NS, 32), jnp.float32)],
        (cnt2, S1, S2))

    C3 = _sc_segsum(src, NS, 32, 3, data=msg, mean=mean)
    C4 = _sc_segsum(src, NS, 32, 4, data=msg, mean=mean)

    slin, ssums = _tc(
        _src_node_body, (gn,),
        [_full((1, _D)), _full((160, 160)), _full((1, 160)), _full((160, _D)),
         _full((1, _D)), _rows(_NB, _D), _rows3(_NB, _D),
         _rows(_NB, 32), _rows(_NB, 32), _rows3(_NB, 32), _rows3(_NB, 32)],
        [_rows(_NB, _D), _sum_spec()],
        [jax.ShapeDtypeStruct((NS, _D), jnp.float32),
         jax.ShapeDtypeStruct((2, _D), jnp.float32)],
        (x_u, p['Ws21'], r2(p['bs21']), p['Ws22'], r2(p['bs22']),
         x_s, cnt2, mean, std, C3, C4))
    m_s = ssums[0] / NS
    v_s = ssums[1] / NS - m_s * m_s
    sc_s = p['g_s'] / jnp.sqrt(v_s + 1e-5)
    sh_s = p['b_s'] - m_s * sc_s

    xsn, s_sum = _tc(
        _fin_body, (gn,),
        [_full((1, _D)), _full((1, _D)), _rows(_NB, _D)],
        [_rows(_NB, _D), pl.BlockSpec((1, _D), lambda i: (0, 0))],
        [jax.ShapeDtypeStruct((NS, _D), jnp.float32),
         jax.ShapeDtypeStruct((1, _D), jnp.float32)],
        (r2(sc_s), r2(sh_s), slin))

    (hss,) = _sc_gather([(xsn, src)])

    (msgt,) = _tc(
        _msg_t_body, (ge,),
        [_full((32, 32)), _full((1, 32)), _full((32, 32)), _full((1, 32)),
         _rows(_EB, _D), _rows(_EB, _D)],
        [_rows(_EB, 32)],
        [jax.ShapeDtypeStruct((E, 32), jnp.float32)],
        (p['Wt11'], r2(p['bt11']), p['Wt12'], r2(p['bt12']), hss, xen))

    agg = _sc_segsum(tgt, NT, 32, 1, data=msgt)

    gt = NT // _NB
    tlin, tsums = _tc(
        _tgt_node_body, (gt,),
        [_full((1, _D)), _full((64, 64)), _full((1, 64)), _full((64, _D)),
         _full((1, _D)), _rows(_NB, _D), _rows3(_NB, 32)],
        [_rows(_NB, _D), _sum_spec()],
        [jax.ShapeDtypeStruct((NT, _D), jnp.float32),
         jax.ShapeDtypeStruct((2, _D), jnp.float32)],
        (x_u, p['Wt21'], r2(p['bt21']), p['Wt22'], r2(p['bt22']), x_t, agg))
    m_t = tsums[0] / NT
    v_t = tsums[1] / NT - m_t * m_t
    sc_t = p['g_t'] / jnp.sqrt(v_t + 1e-5)
    sh_t = p['b_t'] - m_t * sc_t

    xtn, t_sum = _tc(
        _fin_body, (gt,),
        [_full((1, _D)), _full((1, _D)), _rows(_NB, _D)],
        [_rows(_NB, _D), pl.BlockSpec((1, _D), lambda i: (0, 0))],
        [jax.ShapeDtypeStruct((NT, _D), jnp.float32),
         jax.ShapeDtypeStruct((1, _D), jnp.float32)],
        (r2(sc_t), r2(sh_t), tlin))

    (xun,) = _tc(
        functools.partial(_global_body, NS, NT), (1,),
        [_full((1, _D)), _full((1, _D)), _full((1, _D)), _full((48, 48)),
         _full((1, 48)), _full((48, _D)), _full((1, _D)), _full((1, _D))],
        [_full((1, _D))],
        [jax.ShapeDtypeStruct((1, _D), jnp.float32)],
        (s_sum, t_sum, x_u, p['Wg1'], r2(p['bg1']), p['Wg2'], r2(p['bg2']),
         r2(p['g_u'])))

    return (edge_index, xsn, xtn, xen, xun)
